# baseline (speedup 1.0000x reference)
import jax
import jax.numpy as jnp
from jax import lax
from jax.experimental import pallas as pl
from jax.experimental.pallas import tpu as pltpu

_B, _E, _HW, _K = 16, 64, 1024, 1024


_BPB = 2


def _vq_body(x_ref, cb_ref, out_ref):
    cb = cb_ref[...]
    sq2 = jnp.sum(cb * cb, axis=1)[:, None]
    cb2 = cb + cb
    iota = lax.broadcasted_iota(jnp.int32, (_K, _HW), 0)
    for j in range(_BPB):
        x = x_ref[j].reshape(_E, _HW)
        sq1 = jnp.sum(x * x, axis=0)[None, :]
        cross2 = lax.dot_general(cb2, x, (((1,), (0,)), ((), ())),
                                 preferred_element_type=jnp.float32)
        dists = (sq1 - cross2) + sq2
        m = jnp.min(dists, axis=0, keepdims=True)
        idx = jnp.min(jnp.where(dists == m, iota, _K), axis=0)
        onehot = (iota == idx[None, :]).astype(jnp.float32)
        dec = lax.dot_general(cb, onehot, (((0,), (0,)), ((), ())),
                              preferred_element_type=jnp.float32)
        out_ref[j] = dec.reshape(_E, 32, 32)


_vq_call = pl.pallas_call(
    _vq_body,
    grid=(_B // _BPB,),
    in_specs=[
        pl.BlockSpec((_BPB, _E, 32, 32), lambda b: (b, 0, 0, 0)),
        pl.BlockSpec((_K, _E), lambda b: (0, 0)),
    ],
    out_specs=pl.BlockSpec((_BPB, _E, 32, 32), lambda b: (b, 0, 0, 0)),
    out_shape=jax.ShapeDtypeStruct((_B, _E, 32, 32), jnp.float32),
    compiler_params=pltpu.CompilerParams(dimension_semantics=("parallel",)),
)


def kernel(embeddings, codebook):
    return _vq_call(embeddings, codebook)


# speedup vs baseline: 1.0238x; 1.0238x over previous
import jax
import jax.numpy as jnp
from jax import lax
from jax.experimental import pallas as pl
from jax.experimental.pallas import tpu as pltpu

_B, _E, _HW, _K = 16, 64, 1024, 1024


_BPB = 2


def _vq_body(x_ref, cb_ref, out_ref):
    cb = cb_ref[...]
    sq2 = jnp.sum(cb * cb, axis=1)[:, None]
    cb2 = cb + cb
    iota = lax.broadcasted_iota(jnp.int32, (_K, _HW), 0)
    iotaf = iota.astype(jnp.float32)
    for j in range(_BPB):
        x = x_ref[j].reshape(_E, _HW)
        sq1 = jnp.sum(x * x, axis=0)[None, :]
        cross2 = lax.dot_general(cb2, x, (((1,), (0,)), ((), ())),
                                 preferred_element_type=jnp.float32)
        dists = (sq1 - cross2) + sq2
        m = jnp.min(dists, axis=0, keepdims=True)
        idxf = jnp.min(jnp.where(dists == m, iotaf, float(_K)), axis=0)
        onehot = (iotaf == idxf[None, :]).astype(jnp.float32)
        dec = lax.dot_general(cb, onehot, (((0,), (0,)), ((), ())),
                              preferred_element_type=jnp.float32)
        out_ref[j] = dec.reshape(_E, 32, 32)


_vq_call = pl.pallas_call(
    _vq_body,
    grid=(_B // _BPB,),
    in_specs=[
        pl.BlockSpec((_BPB, _E, 32, 32), lambda b: (b, 0, 0, 0)),
        pl.BlockSpec((_K, _E), lambda b: (0, 0)),
    ],
    out_specs=pl.BlockSpec((_BPB, _E, 32, 32), lambda b: (b, 0, 0, 0)),
    out_shape=jax.ShapeDtypeStruct((_B, _E, 32, 32), jnp.float32),
    compiler_params=pltpu.CompilerParams(dimension_semantics=("parallel",)),
)


def kernel(embeddings, codebook):
    return _vq_call(embeddings, codebook)


# speedup vs baseline: 1.1285x; 1.1023x over previous
import jax
import jax.numpy as jnp
from jax import lax
from jax.experimental import pallas as pl
from jax.experimental.pallas import tpu as pltpu

_B, _E, _HW, _K = 16, 64, 1024, 1024


_BPB = 2


_G = 8


def _vq_body(x_ref, cb_ref, out_ref):
    cb = cb_ref[...]
    sq2 = jnp.sum(cb * cb, axis=1)[:, None]
    sq2b = jnp.broadcast_to(sq2, (_K, _HW))
    cb2 = cb + cb
    iotaf = lax.broadcasted_iota(jnp.int32, (_K, _HW), 0).astype(jnp.float32)
    siotaf = lax.broadcasted_iota(jnp.int32, (_G, _HW), 0).astype(jnp.float32)
    for j in range(_BPB):
        x = x_ref[j].reshape(_E, _HW)
        sq1 = jnp.sum(x * x, axis=0)[None, :]
        sq1b = jnp.broadcast_to(sq1, (_G, _HW))
        cross2 = lax.dot_general(cb2, x, (((1,), (0,)), ((), ())),
                                 preferred_element_type=jnp.float32)
        val = (sq1b - cross2[0:_G]) + sq2b[0:_G]
        grp = jnp.zeros((_G, _HW), jnp.float32)
        for r in range(1, _K // _G):
            cur = (sq1b - cross2[r * _G:(r + 1) * _G]) + sq2b[r * _G:(r + 1) * _G]
            mask = cur < val
            grp = jnp.where(mask, float(r), grp)
            val = jnp.minimum(cur, val)
        jf = grp * float(_G) + siotaf
        m1 = jnp.min(val, axis=0, keepdims=True)
        idxf = jnp.min(jnp.where(val == m1, jf, float(_K)), axis=0)
        onehot = (iotaf == idxf[None, :]).astype(jnp.float32)
        dec = lax.dot_general(cb, onehot, (((0,), (0,)), ((), ())),
                              preferred_element_type=jnp.float32)
        out_ref[j] = dec.reshape(_E, 32, 32)


_vq_call = pl.pallas_call(
    _vq_body,
    grid=(_B // _BPB,),
    in_specs=[
        pl.BlockSpec((_BPB, _E, 32, 32), lambda b: (b, 0, 0, 0)),
        pl.BlockSpec((_K, _E), lambda b: (0, 0)),
    ],
    out_specs=pl.BlockSpec((_BPB, _E, 32, 32), lambda b: (b, 0, 0, 0)),
    out_shape=jax.ShapeDtypeStruct((_B, _E, 32, 32), jnp.float32),
    compiler_params=pltpu.CompilerParams(dimension_semantics=("parallel",)),
)


def kernel(embeddings, codebook):
    return _vq_call(embeddings, codebook)
